# Initial kernel scaffold; baseline (speedup 1.0000x reference)
import jax
import jax.numpy as jnp
from jax.experimental import pallas as pl


def kernel(x, dw1, pw1, dw2, pw2):
    raise NotImplementedError("write your pallas kernel here")


# speedup vs baseline: 2.3551x; 2.3551x over previous
import functools

import jax
import jax.numpy as jnp
from jax import lax
from jax.experimental import pallas as pl
from jax.experimental.pallas import tpu as pltpu

_VMEM_LIMIT = 40 * 1024 * 1024


def _stage_core(xt, scale, shift, dww_ref, pww_ref, y_ref, stats_ref, xp_ref,
                *, h, w, k, p):
    c_in = xt.shape[1]
    hp, wp = h + 2 * p, w + 2 * p
    if scale is not None:
        xt = xt * scale + shift
    xt = jnp.maximum(xt, 0.0)

    zrow = jnp.zeros((p, wp, c_in), jnp.float32)
    xp_ref[0:p, :, :] = zrow
    xp_ref[h + p:hp, :, :] = zrow
    zcol = jnp.zeros((h, p, c_in), jnp.float32)
    xp_ref[p:p + h, 0:p, :] = zcol
    xp_ref[p:p + h, w + p:wp, :] = zcol
    xp_ref[p:p + h, p:p + w, :] = xt.reshape(h, w, c_in)

    dww = dww_ref[...]
    acc = None
    for kh in range(k):
        for kw in range(k):
            term = xp_ref[kh:kh + h, kw:kw + w, :] * dww[kh, kw, :]
            acc = term if acc is None else acc + term

    flat = acc.reshape(h * w, c_in).astype(jnp.bfloat16)
    y2d = jnp.dot(flat, pww_ref[...], preferred_element_type=jnp.float32)

    stats_ref[0, 0:1, :] = jnp.sum(y2d, axis=0, keepdims=True)
    stats_ref[0, 1:2, :] = jnp.sum(y2d * y2d, axis=0, keepdims=True)
    y_ref[0] = y2d.astype(y_ref.dtype)


def _stage1_kernel(x_ref, dww_ref, pww_ref, y_ref, stats_ref, xp_ref,
                   *, h, w, k, p):
    xt = jnp.transpose(x_ref[0], (1, 0))
    _stage_core(xt, None, None, dww_ref, pww_ref, y_ref, stats_ref, xp_ref,
                h=h, w=w, k=k, p=p)


def _stage2_kernel(x_ref, scale_ref, shift_ref, dww_ref, pww_ref,
                   y_ref, stats_ref, xp_ref, *, h, w, k, p):
    xt = x_ref[0].astype(jnp.float32)
    _stage_core(xt, scale_ref[...], shift_ref[...], dww_ref, pww_ref,
                y_ref, stats_ref, xp_ref, h=h, w=w, k=k, p=p)


def _bn_transpose_kernel(y_ref, scale_ref, shift_ref, o_ref):
    y = y_ref[0].astype(jnp.float32) * scale_ref[...] + shift_ref[...]
    o_ref[0] = jnp.transpose(y, (1, 0))


def _finalize_stats(stats, count, eps):
    s = jnp.sum(stats[:, 0, :], axis=0)
    sq = jnp.sum(stats[:, 1, :], axis=0)
    mean = s / count
    var = jnp.maximum(sq / count - mean * mean, 0.0)
    scale = lax.rsqrt(var + eps)
    shift = -mean * scale
    return scale.reshape(1, -1), shift.reshape(1, -1)


def kernel(x, dw1, pw1, dw2, pw2, *, eps=1e-5):
    n, c_in, h, w = x.shape
    k = dw1.shape[0]
    c_out = pw2.shape[1]
    p = 1
    hw = h * w
    hp, wp = h + 2 * p, w + 2 * p

    x3 = x.reshape(n, c_in, hw)
    pw1b = pw1.astype(jnp.bfloat16)
    pw2b = pw2.astype(jnp.bfloat16)

    y1, stats1 = pl.pallas_call(
        functools.partial(_stage1_kernel, h=h, w=w, k=k, p=p),
        out_shape=(jax.ShapeDtypeStruct((n, hw, c_in), jnp.bfloat16),
                   jax.ShapeDtypeStruct((n, 2, c_in), jnp.float32)),
        grid=(n,),
        in_specs=[pl.BlockSpec((1, c_in, hw), lambda i: (i, 0, 0)),
                  pl.BlockSpec((k, k, c_in), lambda i: (0, 0, 0)),
                  pl.BlockSpec((c_in, c_in), lambda i: (0, 0))],
        out_specs=(pl.BlockSpec((1, hw, c_in), lambda i: (i, 0, 0)),
                   pl.BlockSpec((1, 2, c_in), lambda i: (i, 0, 0))),
        scratch_shapes=[pltpu.VMEM((hp, wp, c_in), jnp.float32)],
        compiler_params=pltpu.CompilerParams(
            dimension_semantics=("parallel",),
            vmem_limit_bytes=_VMEM_LIMIT),
    )(x3, dw1, pw1b)
    scale1, shift1 = _finalize_stats(stats1, n * hw, eps)

    y2, stats2 = pl.pallas_call(
        functools.partial(_stage2_kernel, h=h, w=w, k=k, p=p),
        out_shape=(jax.ShapeDtypeStruct((n, hw, c_out), jnp.bfloat16),
                   jax.ShapeDtypeStruct((n, 2, c_out), jnp.float32)),
        grid=(n,),
        in_specs=[pl.BlockSpec((1, hw, c_in), lambda i: (i, 0, 0)),
                  pl.BlockSpec((1, c_in), lambda i: (0, 0)),
                  pl.BlockSpec((1, c_in), lambda i: (0, 0)),
                  pl.BlockSpec((k, k, c_in), lambda i: (0, 0, 0)),
                  pl.BlockSpec((c_in, c_out), lambda i: (0, 0))],
        out_specs=(pl.BlockSpec((1, hw, c_out), lambda i: (i, 0, 0)),
                   pl.BlockSpec((1, 2, c_out), lambda i: (i, 0, 0))),
        scratch_shapes=[pltpu.VMEM((hp, wp, c_in), jnp.float32)],
        compiler_params=pltpu.CompilerParams(
            dimension_semantics=("parallel",),
            vmem_limit_bytes=_VMEM_LIMIT),
    )(y1, scale1, shift1, dw2, pw2b)
    scale2, shift2 = _finalize_stats(stats2, n * hw, eps)

    out = pl.pallas_call(
        _bn_transpose_kernel,
        out_shape=jax.ShapeDtypeStruct((n, c_out, hw), jnp.float32),
        grid=(n,),
        in_specs=[pl.BlockSpec((1, hw, c_out), lambda i: (i, 0, 0)),
                  pl.BlockSpec((1, c_out), lambda i: (0, 0)),
                  pl.BlockSpec((1, c_out), lambda i: (0, 0))],
        out_specs=pl.BlockSpec((1, c_out, hw), lambda i: (i, 0, 0)),
        compiler_params=pltpu.CompilerParams(
            dimension_semantics=("parallel",),
            vmem_limit_bytes=_VMEM_LIMIT),
    )(y2, scale2, shift2)
    return out.reshape(n, c_out, h, w)
